# Initial kernel scaffold; baseline (speedup 1.0000x reference)
import jax
import jax.numpy as jnp
from jax.experimental import pallas as pl


def kernel(x_ap, x_user, x_target, ea_s, ea_tx, ea_rx, params, ei_s, ei_tx, ei_rx):
    raise NotImplementedError("write your pallas kernel here")


# speedup vs baseline: 1.5812x; 1.5812x over previous
import functools

import jax
import jax.numpy as jnp
from jax.experimental import pallas as pl
from jax.experimental.pallas import tpu as pltpu

H = 96
BE = 256


def _mlp(p, x):
    h = jax.nn.relu(x @ p["l1"]["w"] + p["l1"]["b"])
    return h @ p["l2"]["w"] + p["l2"]["b"]


def _layernorm(x, g, b, eps=1e-5):
    m = jnp.mean(x, axis=-1, keepdims=True)
    v = jnp.var(x, axis=-1, keepdims=True)
    return (x - m) / jnp.sqrt(v + eps) * g + b


def _msg_body(ea_ref, xf_ref, xr_ref, w1_ref, b1_ref, w2_ref, b2_ref,
              of_ref, or_ref):
    g = jnp.maximum(
        jnp.dot(ea_ref[...], w1_ref[...], preferred_element_type=jnp.float32)
        + b1_ref[...], 0.0)
    w = (jnp.dot(g, w2_ref[...], preferred_element_type=jnp.float32)
         + b2_ref[...])
    w3 = w.reshape(BE, H, H)
    of_ref[...] = jnp.einsum('eh,eho->eo', xf_ref[...], w3,
                             preferred_element_type=jnp.float32)
    or_ref[...] = jnp.einsum('eh,eho->eo', xr_ref[...], w3,
                             preferred_element_type=jnp.float32)


def _fused_msgs(ea, xf, xr, net):
    e_pad = ea.shape[0]
    de = ea.shape[1]
    w1 = net["l1"]["w"]
    b1 = net["l1"]["b"].reshape(1, H)
    w2 = net["l2"]["w"]
    b2 = net["l2"]["b"].reshape(1, H * H)
    grid = (e_pad // BE,)
    of, orv = pl.pallas_call(
        _msg_body,
        grid=grid,
        in_specs=[
            pl.BlockSpec((BE, de), lambda i: (i, 0)),
            pl.BlockSpec((BE, H), lambda i: (i, 0)),
            pl.BlockSpec((BE, H), lambda i: (i, 0)),
            pl.BlockSpec((de, H), lambda i: (0, 0)),
            pl.BlockSpec((1, H), lambda i: (0, 0)),
            pl.BlockSpec((H, H * H), lambda i: (0, 0)),
            pl.BlockSpec((1, H * H), lambda i: (0, 0)),
        ],
        out_specs=[
            pl.BlockSpec((BE, H), lambda i: (i, 0)),
            pl.BlockSpec((BE, H), lambda i: (i, 0)),
        ],
        out_shape=[
            jax.ShapeDtypeStruct((e_pad, H), jnp.float32),
            jax.ShapeDtypeStruct((e_pad, H), jnp.float32),
        ],
    )(ea, xf, xr, w1, b1, w2, b2)
    return of, orv


def _pad_rows(x, e_pad):
    return jnp.pad(x, ((0, e_pad - x.shape[0]), (0, 0)))


def _seg_mean(msg, dst, n):
    sums = jax.ops.segment_sum(msg, dst, num_segments=n)
    cnt = jax.ops.segment_sum(jnp.ones((dst.shape[0],), msg.dtype), dst,
                              num_segments=n)
    return sums / jnp.maximum(cnt, 1.0)[:, None]


def kernel(x_ap, x_user, x_target, ea_s, ea_tx, ea_rx, params, ei_s, ei_tx,
           ei_rx):
    h_ap = _layernorm(_mlp(params["ap_in"], x_ap),
                      params["ln"]["ap"]["g"], params["ln"]["ap"]["b"])
    h_user = _layernorm(_mlp(params["user_in"], x_user),
                        params["ln"]["user"]["g"], params["ln"]["user"]["b"])
    h_tgt = _layernorm(_mlp(params["tgt_in"], x_target),
                       params["ln"]["tgt"]["g"], params["ln"]["tgt"]["b"])

    e_s = ea_s.shape[0]
    e_tx = ea_tx.shape[0]
    e_rx = ea_rx.shape[0]
    ep_s = (e_s + BE - 1) // BE * BE
    ep_tx = (e_tx + BE - 1) // BE * BE
    ep_rx = (e_rx + BE - 1) // BE * BE
    ea_s_p = _pad_rows(ea_s, ep_s)
    ea_tx_p = _pad_rows(ea_tx, ep_tx)
    ea_rx_p = _pad_rows(ea_rx, ep_rx)

    n_ap, n_user, n_tgt = x_ap.shape[0], x_user.shape[0], x_target.shape[0]

    for lp in params["layers"]:
        c = lp["conv"]
        xs_ap = _pad_rows(h_ap[ei_s[0]], ep_s)
        xs_user = _pad_rows(h_user[ei_s[1]], ep_s)
        xtx_ap = _pad_rows(h_ap[ei_tx[0]], ep_tx)
        xtx_tgt = _pad_rows(h_tgt[ei_tx[1]], ep_tx)
        xrx_ap = _pad_rows(h_ap[ei_rx[0]], ep_rx)
        xrx_tgt = _pad_rows(h_tgt[ei_rx[1]], ep_rx)

        m_s_f, m_s_r = _fused_msgs(ea_s_p, xs_ap, xs_user, lp["net_serv"])
        m_tx_f, m_tx_r = _fused_msgs(ea_tx_p, xtx_ap, xtx_tgt, lp["net_sens"])
        m_rx_f, m_rx_r = _fused_msgs(ea_rx_p, xrx_ap, xrx_tgt, lp["net_sens"])

        agg_user = _seg_mean(m_s_f[:e_s], ei_s[1], n_user)
        agg_ap = (_seg_mean(m_s_r[:e_s], ei_s[0], n_ap)
                  + _seg_mean(m_tx_r[:e_tx], ei_tx[0], n_ap)
                  + _seg_mean(m_rx_r[:e_rx], ei_rx[0], n_ap))
        agg_tgt = (_seg_mean(m_tx_f[:e_tx], ei_tx[1], n_tgt)
                   + _seg_mean(m_rx_f[:e_rx], ei_rx[1], n_tgt))

        out_user = agg_user + h_user @ c["serves"]["root"] + c["serves"]["bias"]
        out_ap = (agg_ap
                  + h_ap @ (c["rev_served"]["root"] + c["rev_tx"]["root"]
                            + c["rev_rx"]["root"])
                  + c["rev_served"]["bias"] + c["rev_tx"]["bias"]
                  + c["rev_rx"]["bias"])
        out_tgt = (agg_tgt
                   + h_tgt @ (c["tx"]["root"] + c["rx"]["root"])
                   + c["tx"]["bias"] + c["rx"]["bias"])
        h_ap = jax.nn.relu(out_ap)
        h_user = jax.nn.relu(out_user)
        h_tgt = jax.nn.relu(out_tgt)

    x_feats = jnp.concatenate([h_ap[ei_s[0]], h_user[ei_s[1]], ea_s], axis=-1)
    x_logit = _mlp(params["edge_head"], x_feats)
    tau_logit = _mlp(params["ap_head"], h_ap)
    s_logit = _mlp(params["tgt_head"], h_tgt)
    return x_logit, tau_logit, s_logit


# speedup vs baseline: 2.2826x; 1.4436x over previous
import functools

import jax
import jax.numpy as jnp
from jax import lax
from jax.experimental import pallas as pl
from jax.experimental.pallas import tpu as pltpu
from jax.experimental.pallas import tpu_sc as plsc

H = 96
BE = 256
NC, NS = 2, 16
CH = 128

NP_USER, NP_AP, NP_TGT = 10240, 2048, 4096
SH = 12288
STRIPE = SH // NS

SET_CORE = (0, 0, 1, 1, 1, 1)
SET_OFF = (0, NP_USER, 0, NP_AP, 2 * NP_AP, 2 * NP_AP + NP_TGT)


def _mlp(p, x):
    h = jax.nn.relu(x @ p["l1"]["w"] + p["l1"]["b"])
    return h @ p["l2"]["w"] + p["l2"]["b"]


def _layernorm(x, g, b, eps=1e-5):
    m = jnp.mean(x, axis=-1, keepdims=True)
    v = jnp.var(x, axis=-1, keepdims=True)
    return (x - m) / jnp.sqrt(v + eps) * g + b


HP = 128


def _msg_body(e_real, ea_ref, xf_ref, xr_ref, w1_ref, b1_ref, w2_ref, b2_ref,
              of_ref, or_ref):
    g = jnp.maximum(
        jnp.dot(ea_ref[...], w1_ref[...], preferred_element_type=jnp.float32)
        + b1_ref[...], 0.0)
    w = (jnp.dot(g, w2_ref[...], preferred_element_type=jnp.float32)
         + b2_ref[...])
    w3 = w.reshape(BE, H, HP)
    rows = pl.program_id(0) * BE + lax.broadcasted_iota(jnp.int32, (BE, 1), 0)
    valid = rows < e_real
    mf = jnp.einsum('eh,eho->eo', xf_ref[...], w3,
                    preferred_element_type=jnp.float32)
    mr = jnp.einsum('eh,eho->eo', xr_ref[...], w3,
                    preferred_element_type=jnp.float32)
    of_ref[...] = jnp.where(valid, mf, 0.0)
    or_ref[...] = jnp.where(valid, mr, 0.0)


def _fused_msgs(ea, xf, xr, net, e_real):
    e_pad = ea.shape[0]
    de = ea.shape[1]
    w1 = net["l1"]["w"]
    b1 = net["l1"]["b"].reshape(1, H)
    w2p = jnp.pad(net["l2"]["w"].reshape(H, H, H),
                  ((0, 0), (0, 0), (0, HP - H))).reshape(H, H * HP)
    b2p = jnp.pad(net["l2"]["b"].reshape(H, H),
                  ((0, 0), (0, HP - H))).reshape(1, H * HP)
    grid = (e_pad // BE,)
    of, orv = pl.pallas_call(
        functools.partial(_msg_body, e_real),
        grid=grid,
        in_specs=[
            pl.BlockSpec((BE, de), lambda i: (i, 0)),
            pl.BlockSpec((BE, H), lambda i: (i, 0)),
            pl.BlockSpec((BE, H), lambda i: (i, 0)),
            pl.BlockSpec((de, H), lambda i: (0, 0)),
            pl.BlockSpec((1, H), lambda i: (0, 0)),
            pl.BlockSpec((H, H * HP), lambda i: (0, 0)),
            pl.BlockSpec((1, H * HP), lambda i: (0, 0)),
        ],
        out_specs=[
            pl.BlockSpec((BE, HP), lambda i: (i, 0)),
            pl.BlockSpec((BE, HP), lambda i: (i, 0)),
        ],
        out_shape=[
            jax.ShapeDtypeStruct((e_pad, HP), jnp.float32),
            jax.ShapeDtypeStruct((e_pad, HP), jnp.float32),
        ],
    )(ea, xf, xr, w1, b1, w2p, b2p)
    return of, orv


def _scatter_body(eps, width, *refs):
    msgs = refs[0:6]
    idxs = refs[6:12]
    zeros = refs[12]
    out = refs[13]
    vbuf = refs[14]
    ibuf = refs[15]
    sh = refs[16]
    cid = lax.axis_index("c")
    sid = lax.axis_index("s")
    pltpu.sync_copy(zeros, sh.at[pl.ds(sid * STRIPE, STRIPE)])
    plsc.subcore_barrier()
    for k in range(6):
        nch = eps[k] // CH
        for j in range(-(-nch // NS)):
            c = sid + NS * j

            @pl.when(jnp.logical_and(cid == SET_CORE[k], c < nch))
            def _():
                pltpu.sync_copy(idxs[k].at[c], ibuf)
                pltpu.sync_copy(msgs[k].at[pl.ds(c * CH, CH)], vbuf)
                pltpu.sync_copy(vbuf, sh.at[ibuf], add=True)
    plsc.subcore_barrier()
    pltpu.sync_copy(sh.at[pl.ds(sid * STRIPE, STRIPE)],
                    out.at[cid].at[pl.ds(sid * STRIPE, STRIPE)])


def _sc_scatter(msgs, idx2ds, width):
    eps = tuple(m.shape[0] for m in msgs)
    mesh = plsc.VectorSubcoreMesh(core_axis_name="c", subcore_axis_name="s",
                                  num_cores=NC, num_subcores=NS)
    zeros = jnp.zeros((STRIPE, width), jnp.float32)
    out = pl.kernel(
        functools.partial(_scatter_body, eps, width),
        out_type=jax.ShapeDtypeStruct((NC, SH, width), jnp.float32),
        mesh=mesh,
        scratch_types=[
            pltpu.VMEM((CH, width), jnp.float32),
            pltpu.VMEM((CH,), jnp.int32),
            pltpu.VMEM_SHARED((SH, width), jnp.float32),
        ],
    )(*msgs, *idx2ds, zeros)
    return out


def _prep_idx(idx, e_pad, off):
    idx = jnp.pad(idx, (0, e_pad - idx.shape[0])) + off
    return idx.reshape(e_pad // CH, CH).astype(jnp.int32)


def _pad_rows(x, e_pad):
    return jnp.pad(x, ((0, e_pad - x.shape[0]), (0, 0)))


def kernel(x_ap, x_user, x_target, ea_s, ea_tx, ea_rx, params, ei_s, ei_tx,
           ei_rx):
    h_ap = _layernorm(_mlp(params["ap_in"], x_ap),
                      params["ln"]["ap"]["g"], params["ln"]["ap"]["b"])
    h_user = _layernorm(_mlp(params["user_in"], x_user),
                        params["ln"]["user"]["g"], params["ln"]["user"]["b"])
    h_tgt = _layernorm(_mlp(params["tgt_in"], x_target),
                       params["ln"]["tgt"]["g"], params["ln"]["tgt"]["b"])

    e_s, e_tx, e_rx = ea_s.shape[0], ea_tx.shape[0], ea_rx.shape[0]
    ep_s = (e_s + BE - 1) // BE * BE
    ep_tx = (e_tx + BE - 1) // BE * BE
    ep_rx = (e_rx + BE - 1) // BE * BE
    ea_s_p = _pad_rows(ea_s, ep_s)
    ea_tx_p = _pad_rows(ea_tx, ep_tx)
    ea_rx_p = _pad_rows(ea_rx, ep_rx)

    n_ap, n_user, n_tgt = x_ap.shape[0], x_user.shape[0], x_target.shape[0]

    idx2ds = (
        _prep_idx(ei_s[1], ep_s, SET_OFF[0]),
        _prep_idx(ei_s[0], ep_s, SET_OFF[1]),
        _prep_idx(ei_tx[0], ep_tx, SET_OFF[2]),
        _prep_idx(ei_rx[0], ep_rx, SET_OFF[3]),
        _prep_idx(ei_tx[1], ep_tx, SET_OFF[4]),
        _prep_idx(ei_rx[1], ep_rx, SET_OFF[5]),
    )

    def ones_masked(e_pad, e_real):
        return jnp.where(jnp.arange(e_pad)[:, None] < e_real,
                         jnp.float32(1), jnp.float32(0)) * jnp.ones((1, HP),
                                                                    jnp.float32)
    cnt = _sc_scatter(
        (ones_masked(ep_s, e_s), ones_masked(ep_s, e_s),
         ones_masked(ep_tx, e_tx), ones_masked(ep_rx, e_rx),
         ones_masked(ep_tx, e_tx), ones_masked(ep_rx, e_rx)),
        idx2ds, HP)
    inv = 1.0 / jnp.maximum(cnt[:, :, :1], 1.0)
    inv_user = inv[0, :n_user]
    inv_ap = (inv[0, NP_USER:NP_USER + n_ap], inv[1, :n_ap],
              inv[1, NP_AP:NP_AP + n_ap])
    inv_tgt = (inv[1, 2 * NP_AP:2 * NP_AP + n_tgt],
               inv[1, 2 * NP_AP + NP_TGT:2 * NP_AP + NP_TGT + n_tgt])

    for lp in params["layers"]:
        c = lp["conv"]
        xs_ap = _pad_rows(h_ap[ei_s[0]], ep_s)
        xs_user = _pad_rows(h_user[ei_s[1]], ep_s)
        xtx_ap = _pad_rows(h_ap[ei_tx[0]], ep_tx)
        xtx_tgt = _pad_rows(h_tgt[ei_tx[1]], ep_tx)
        xrx_ap = _pad_rows(h_ap[ei_rx[0]], ep_rx)
        xrx_tgt = _pad_rows(h_tgt[ei_rx[1]], ep_rx)

        m_s_f, m_s_r = _fused_msgs(ea_s_p, xs_ap, xs_user, lp["net_serv"], e_s)
        m_tx_f, m_tx_r = _fused_msgs(ea_tx_p, xtx_ap, xtx_tgt,
                                     lp["net_sens"], e_tx)
        m_rx_f, m_rx_r = _fused_msgs(ea_rx_p, xrx_ap, xrx_tgt,
                                     lp["net_sens"], e_rx)

        p = _sc_scatter((m_s_f, m_s_r, m_tx_r, m_rx_r, m_tx_f, m_rx_f),
                        idx2ds, HP)[:, :, :H]
        agg_user = p[0, :n_user] * inv_user
        agg_ap = (p[0, NP_USER:NP_USER + n_ap] * inv_ap[0]
                  + p[1, :n_ap] * inv_ap[1]
                  + p[1, NP_AP:NP_AP + n_ap] * inv_ap[2])
        agg_tgt = (p[1, 2 * NP_AP:2 * NP_AP + n_tgt] * inv_tgt[0]
                   + p[1, 2 * NP_AP + NP_TGT:2 * NP_AP + NP_TGT + n_tgt]
                   * inv_tgt[1])

        out_user = agg_user + h_user @ c["serves"]["root"] + c["serves"]["bias"]
        out_ap = (agg_ap
                  + h_ap @ (c["rev_served"]["root"] + c["rev_tx"]["root"]
                            + c["rev_rx"]["root"])
                  + c["rev_served"]["bias"] + c["rev_tx"]["bias"]
                  + c["rev_rx"]["bias"])
        out_tgt = (agg_tgt
                   + h_tgt @ (c["tx"]["root"] + c["rx"]["root"])
                   + c["tx"]["bias"] + c["rx"]["bias"])
        h_ap = jax.nn.relu(out_ap)
        h_user = jax.nn.relu(out_user)
        h_tgt = jax.nn.relu(out_tgt)

    x_feats = jnp.concatenate([h_ap[ei_s[0]], h_user[ei_s[1]], ea_s], axis=-1)
    x_logit = _mlp(params["edge_head"], x_feats)
    tau_logit = _mlp(params["ap_head"], h_ap)
    s_logit = _mlp(params["tgt_head"], h_tgt)
    return x_logit, tau_logit, s_logit


# speedup vs baseline: 2.3644x; 1.0358x over previous
import functools

import jax
import jax.numpy as jnp
from jax import lax
from jax.experimental import pallas as pl
from jax.experimental.pallas import tpu as pltpu
from jax.experimental.pallas import tpu_sc as plsc

H = 96
BE = 256
NC, NS = 2, 16
CH = 128

NP_USER, NP_AP, NP_TGT = 10240, 2048, 4096
SH = 12288
STRIPE = SH // NS

SET_CORE = (0, 0, 1, 1, 1, 1)
SET_OFF = (0, NP_USER, 0, NP_AP, 2 * NP_AP, 2 * NP_AP + NP_TGT)


def _mlp(p, x):
    h = jax.nn.relu(x @ p["l1"]["w"] + p["l1"]["b"])
    return h @ p["l2"]["w"] + p["l2"]["b"]


def _layernorm(x, g, b, eps=1e-5):
    m = jnp.mean(x, axis=-1, keepdims=True)
    v = jnp.var(x, axis=-1, keepdims=True)
    return (x - m) / jnp.sqrt(v + eps) * g + b


HP = 128


def _msg_body(e_real, ea_ref, xf_ref, xr_ref, w1_ref, b1_ref, w2_ref, b2_ref,
              of_ref, or_ref):
    g = jnp.maximum(
        jnp.dot(ea_ref[...], w1_ref[...], preferred_element_type=jnp.float32)
        + b1_ref[...], 0.0)
    w = (jnp.dot(g.astype(jnp.bfloat16), w2_ref[...],
                 preferred_element_type=jnp.float32)
         + b2_ref[...]).astype(jnp.bfloat16)
    w3 = w.reshape(BE, H, HP)
    rows = pl.program_id(0) * BE + lax.broadcasted_iota(jnp.int32, (BE, 1), 0)
    valid = rows < e_real
    mf = jnp.einsum('eh,eho->eo', xf_ref[...].astype(jnp.bfloat16), w3,
                    preferred_element_type=jnp.float32)
    mr = jnp.einsum('eh,eho->eo', xr_ref[...].astype(jnp.bfloat16), w3,
                    preferred_element_type=jnp.float32)
    of_ref[...] = jnp.where(valid, mf, 0.0)
    or_ref[...] = jnp.where(valid, mr, 0.0)


def _fused_msgs(ea, xf, xr, net, e_real):
    e_pad = ea.shape[0]
    de = ea.shape[1]
    w1 = net["l1"]["w"]
    b1 = net["l1"]["b"].reshape(1, H)
    w2p = jnp.pad(net["l2"]["w"].reshape(H, H, H),
                  ((0, 0), (0, 0), (0, HP - H))).reshape(H, H * HP)
    b2p = jnp.pad(net["l2"]["b"].reshape(H, H),
                  ((0, 0), (0, HP - H))).reshape(1, H * HP)
    grid = (e_pad // BE,)
    of, orv = pl.pallas_call(
        functools.partial(_msg_body, e_real),
        grid=grid,
        in_specs=[
            pl.BlockSpec((BE, de), lambda i: (i, 0)),
            pl.BlockSpec((BE, H), lambda i: (i, 0)),
            pl.BlockSpec((BE, H), lambda i: (i, 0)),
            pl.BlockSpec((de, H), lambda i: (0, 0)),
            pl.BlockSpec((1, H), lambda i: (0, 0)),
            pl.BlockSpec((H, H * HP), lambda i: (0, 0)),
            pl.BlockSpec((1, H * HP), lambda i: (0, 0)),
        ],
        out_specs=[
            pl.BlockSpec((BE, HP), lambda i: (i, 0)),
            pl.BlockSpec((BE, HP), lambda i: (i, 0)),
        ],
        out_shape=[
            jax.ShapeDtypeStruct((e_pad, HP), jnp.float32),
            jax.ShapeDtypeStruct((e_pad, HP), jnp.float32),
        ],
    )(ea, xf, xr, w1, b1, w2p.astype(jnp.bfloat16), b2p)
    return of, orv


def _scatter_body(eps, width, *refs):
    msgs = refs[0:6]
    idxs = refs[6:12]
    zeros = refs[12]
    out = refs[13]
    vbuf = refs[14]
    ibuf = refs[15]
    sh = refs[16]
    cid = lax.axis_index("c")
    sid = lax.axis_index("s")
    pltpu.sync_copy(zeros, sh.at[pl.ds(sid * STRIPE, STRIPE)])
    plsc.subcore_barrier()
    for k in range(6):
        nch = eps[k] // CH
        for j in range(-(-nch // NS)):
            c = sid + NS * j

            @pl.when(jnp.logical_and(cid == SET_CORE[k], c < nch))
            def _():
                pltpu.sync_copy(idxs[k].at[c], ibuf)
                pltpu.sync_copy(msgs[k].at[pl.ds(c * CH, CH)], vbuf)
                pltpu.sync_copy(vbuf, sh.at[ibuf], add=True)
    plsc.subcore_barrier()
    pltpu.sync_copy(sh.at[pl.ds(sid * STRIPE, STRIPE)],
                    out.at[cid].at[pl.ds(sid * STRIPE, STRIPE)])


def _sc_scatter(msgs, idx2ds, width):
    eps = tuple(m.shape[0] for m in msgs)
    mesh = plsc.VectorSubcoreMesh(core_axis_name="c", subcore_axis_name="s",
                                  num_cores=NC, num_subcores=NS)
    zeros = jnp.zeros((STRIPE, width), jnp.float32)
    out = pl.kernel(
        functools.partial(_scatter_body, eps, width),
        out_type=jax.ShapeDtypeStruct((NC, SH, width), jnp.float32),
        mesh=mesh,
        scratch_types=[
            pltpu.VMEM((CH, width), jnp.float32),
            pltpu.VMEM((CH,), jnp.int32),
            pltpu.VMEM_SHARED((SH, width), jnp.float32),
        ],
    )(*msgs, *idx2ds, zeros)
    return out


def _prep_idx(idx, e_pad, off):
    idx = jnp.pad(idx, (0, e_pad - idx.shape[0])) + off
    return idx.reshape(e_pad // CH, CH).astype(jnp.int32)


def _pad_rows(x, e_pad):
    return jnp.pad(x, ((0, e_pad - x.shape[0]), (0, 0)))


def kernel(x_ap, x_user, x_target, ea_s, ea_tx, ea_rx, params, ei_s, ei_tx,
           ei_rx):
    h_ap = _layernorm(_mlp(params["ap_in"], x_ap),
                      params["ln"]["ap"]["g"], params["ln"]["ap"]["b"])
    h_user = _layernorm(_mlp(params["user_in"], x_user),
                        params["ln"]["user"]["g"], params["ln"]["user"]["b"])
    h_tgt = _layernorm(_mlp(params["tgt_in"], x_target),
                       params["ln"]["tgt"]["g"], params["ln"]["tgt"]["b"])

    e_s, e_tx, e_rx = ea_s.shape[0], ea_tx.shape[0], ea_rx.shape[0]
    ep_s = (e_s + BE - 1) // BE * BE
    ep_tx = (e_tx + BE - 1) // BE * BE
    ep_rx = (e_rx + BE - 1) // BE * BE
    ea_s_p = _pad_rows(ea_s, ep_s)
    ea_tx_p = _pad_rows(ea_tx, ep_tx)
    ea_rx_p = _pad_rows(ea_rx, ep_rx)

    n_ap, n_user, n_tgt = x_ap.shape[0], x_user.shape[0], x_target.shape[0]

    idx2ds = (
        _prep_idx(ei_s[1], ep_s, SET_OFF[0]),
        _prep_idx(ei_s[0], ep_s, SET_OFF[1]),
        _prep_idx(ei_tx[0], ep_tx, SET_OFF[2]),
        _prep_idx(ei_rx[0], ep_rx, SET_OFF[3]),
        _prep_idx(ei_tx[1], ep_tx, SET_OFF[4]),
        _prep_idx(ei_rx[1], ep_rx, SET_OFF[5]),
    )

    def ones_masked(e_pad, e_real):
        return jnp.where(jnp.arange(e_pad)[:, None] < e_real,
                         jnp.float32(1), jnp.float32(0)) * jnp.ones((1, HP),
                                                                    jnp.float32)
    cnt = _sc_scatter(
        (ones_masked(ep_s, e_s), ones_masked(ep_s, e_s),
         ones_masked(ep_tx, e_tx), ones_masked(ep_rx, e_rx),
         ones_masked(ep_tx, e_tx), ones_masked(ep_rx, e_rx)),
        idx2ds, HP)
    inv = 1.0 / jnp.maximum(cnt[:, :, :1], 1.0)
    inv_user = inv[0, :n_user]
    inv_ap = (inv[0, NP_USER:NP_USER + n_ap], inv[1, :n_ap],
              inv[1, NP_AP:NP_AP + n_ap])
    inv_tgt = (inv[1, 2 * NP_AP:2 * NP_AP + n_tgt],
               inv[1, 2 * NP_AP + NP_TGT:2 * NP_AP + NP_TGT + n_tgt])

    for lp in params["layers"]:
        c = lp["conv"]
        xs_ap = _pad_rows(h_ap[ei_s[0]], ep_s)
        xs_user = _pad_rows(h_user[ei_s[1]], ep_s)
        xtx_ap = _pad_rows(h_ap[ei_tx[0]], ep_tx)
        xtx_tgt = _pad_rows(h_tgt[ei_tx[1]], ep_tx)
        xrx_ap = _pad_rows(h_ap[ei_rx[0]], ep_rx)
        xrx_tgt = _pad_rows(h_tgt[ei_rx[1]], ep_rx)

        m_s_f, m_s_r = _fused_msgs(ea_s_p, xs_ap, xs_user, lp["net_serv"], e_s)
        m_tx_f, m_tx_r = _fused_msgs(ea_tx_p, xtx_ap, xtx_tgt,
                                     lp["net_sens"], e_tx)
        m_rx_f, m_rx_r = _fused_msgs(ea_rx_p, xrx_ap, xrx_tgt,
                                     lp["net_sens"], e_rx)

        p = _sc_scatter((m_s_f, m_s_r, m_tx_r, m_rx_r, m_tx_f, m_rx_f),
                        idx2ds, HP)[:, :, :H]
        agg_user = p[0, :n_user] * inv_user
        agg_ap = (p[0, NP_USER:NP_USER + n_ap] * inv_ap[0]
                  + p[1, :n_ap] * inv_ap[1]
                  + p[1, NP_AP:NP_AP + n_ap] * inv_ap[2])
        agg_tgt = (p[1, 2 * NP_AP:2 * NP_AP + n_tgt] * inv_tgt[0]
                   + p[1, 2 * NP_AP + NP_TGT:2 * NP_AP + NP_TGT + n_tgt]
                   * inv_tgt[1])

        out_user = agg_user + h_user @ c["serves"]["root"] + c["serves"]["bias"]
        out_ap = (agg_ap
                  + h_ap @ (c["rev_served"]["root"] + c["rev_tx"]["root"]
                            + c["rev_rx"]["root"])
                  + c["rev_served"]["bias"] + c["rev_tx"]["bias"]
                  + c["rev_rx"]["bias"])
        out_tgt = (agg_tgt
                   + h_tgt @ (c["tx"]["root"] + c["rx"]["root"])
                   + c["tx"]["bias"] + c["rx"]["bias"])
        h_ap = jax.nn.relu(out_ap)
        h_user = jax.nn.relu(out_user)
        h_tgt = jax.nn.relu(out_tgt)

    x_feats = jnp.concatenate([h_ap[ei_s[0]], h_user[ei_s[1]], ea_s], axis=-1)
    x_logit = _mlp(params["edge_head"], x_feats)
    tau_logit = _mlp(params["ap_head"], h_ap)
    s_logit = _mlp(params["tgt_head"], h_tgt)
    return x_logit, tau_logit, s_logit


# speedup vs baseline: 2.3793x; 1.0063x over previous
import functools

import jax
import jax.numpy as jnp
from jax import lax
from jax.experimental import pallas as pl
from jax.experimental.pallas import tpu as pltpu
from jax.experimental.pallas import tpu_sc as plsc

H = 96
BE = 256
NC, NS = 2, 16
CH = 128

NP_USER, NP_AP, NP_TGT = 10240, 2048, 4096
SH = 12288
STRIPE = SH // NS

SET_CORE = (0, 0, 1, 1, 1, 1)
SET_OFF = (0, NP_USER, 0, NP_AP, 2 * NP_AP, 2 * NP_AP + NP_TGT)


def _mlp(p, x):
    h = jax.nn.relu(x @ p["l1"]["w"] + p["l1"]["b"])
    return h @ p["l2"]["w"] + p["l2"]["b"]


def _layernorm(x, g, b, eps=1e-5):
    m = jnp.mean(x, axis=-1, keepdims=True)
    v = jnp.var(x, axis=-1, keepdims=True)
    return (x - m) / jnp.sqrt(v + eps) * g + b


HP = 128


def _msg_body(e_real, ea_ref, xf_ref, xr_ref, w1_ref, b1_ref, w2_ref, b2_ref,
              of_ref, or_ref):
    g = jnp.maximum(
        jnp.dot(ea_ref[...], w1_ref[...], preferred_element_type=jnp.float32)
        + b1_ref[...], 0.0)
    w = (jnp.dot(g.astype(jnp.bfloat16), w2_ref[...],
                 preferred_element_type=jnp.float32)
         + b2_ref[...]).astype(jnp.bfloat16)
    w3 = w.reshape(BE, H, HP)
    rows = pl.program_id(0) * BE + lax.broadcasted_iota(jnp.int32, (BE, 1), 0)
    valid = rows < e_real
    mf = jnp.einsum('eh,eho->eo',
                    xf_ref[...][:, :H].astype(jnp.bfloat16), w3,
                    preferred_element_type=jnp.float32)
    mr = jnp.einsum('eh,eho->eo',
                    xr_ref[...][:, :H].astype(jnp.bfloat16), w3,
                    preferred_element_type=jnp.float32)
    of_ref[...] = jnp.where(valid, mf, 0.0)
    or_ref[...] = jnp.where(valid, mr, 0.0)


def _fused_msgs(ea, xf, xr, net, e_real):
    e_pad = ea.shape[0]
    de = ea.shape[1]
    w1 = net["l1"]["w"]
    b1 = net["l1"]["b"].reshape(1, H)
    w2p = jnp.pad(net["l2"]["w"].reshape(H, H, H),
                  ((0, 0), (0, 0), (0, HP - H))).reshape(H, H * HP)
    b2p = jnp.pad(net["l2"]["b"].reshape(H, H),
                  ((0, 0), (0, HP - H))).reshape(1, H * HP)
    grid = (e_pad // BE,)
    of, orv = pl.pallas_call(
        functools.partial(_msg_body, e_real),
        grid=grid,
        in_specs=[
            pl.BlockSpec((BE, de), lambda i: (i, 0)),
            pl.BlockSpec((BE, HP), lambda i: (i, 0)),
            pl.BlockSpec((BE, HP), lambda i: (i, 0)),
            pl.BlockSpec((de, H), lambda i: (0, 0)),
            pl.BlockSpec((1, H), lambda i: (0, 0)),
            pl.BlockSpec((H, H * HP), lambda i: (0, 0)),
            pl.BlockSpec((1, H * HP), lambda i: (0, 0)),
        ],
        out_specs=[
            pl.BlockSpec((BE, HP), lambda i: (i, 0)),
            pl.BlockSpec((BE, HP), lambda i: (i, 0)),
        ],
        out_shape=[
            jax.ShapeDtypeStruct((e_pad, HP), jnp.float32),
            jax.ShapeDtypeStruct((e_pad, HP), jnp.float32),
        ],
    )(ea, xf, xr, w1, b1, w2p.astype(jnp.bfloat16), b2p)
    return of, orv


def _gather_body(table_ids, nchs, *refs):
    ntab = max(table_ids) + 1
    nset = len(nchs)
    tables = refs[:ntab]
    idxs = refs[ntab:ntab + nset]
    outs = refs[ntab + nset:ntab + 2 * nset]
    vbuf, ibuf, sem = refs[-3], refs[-2], refs[-1]
    cid = lax.axis_index("c")
    sid = lax.axis_index("s")
    wid = sid * NC + cid
    nw = NC * NS
    for k in range(nset):
        nch = nchs[k]
        for j in range(-(-nch // nw)):
            c = wid + nw * j

            @pl.when(c < nch)
            def _():
                pltpu.sync_copy(idxs[k].at[c], ibuf)
                pltpu.async_copy(tables[table_ids[k]].at[ibuf], vbuf,
                                 sem).wait()
                pltpu.sync_copy(vbuf, outs[k].at[pl.ds(c * CH, CH)])


def _sc_gather(tables, table_ids, idx2ds):
    nchs = tuple(ix.shape[0] for ix in idx2ds)
    mesh = plsc.VectorSubcoreMesh(core_axis_name="c", subcore_axis_name="s",
                                  num_cores=NC, num_subcores=NS)
    outs = pl.kernel(
        functools.partial(_gather_body, table_ids, nchs),
        out_type=[jax.ShapeDtypeStruct((n * CH, HP), jnp.float32)
                  for n in nchs],
        mesh=mesh,
        scratch_types=[
            pltpu.VMEM((CH, HP), jnp.float32),
            pltpu.VMEM((CH,), jnp.int32),
            pltpu.SemaphoreType.DMA,
        ],
    )(*tables, *idx2ds)
    return outs


def _scatter_body(eps, width, *refs):
    msgs = refs[0:6]
    idxs = refs[6:12]
    zeros = refs[12]
    out = refs[13]
    vbuf = refs[14]
    ibuf = refs[15]
    sh = refs[16]
    cid = lax.axis_index("c")
    sid = lax.axis_index("s")
    pltpu.sync_copy(zeros, sh.at[pl.ds(sid * STRIPE, STRIPE)])
    plsc.subcore_barrier()
    for k in range(6):
        nch = eps[k] // CH
        for j in range(-(-nch // NS)):
            c = sid + NS * j

            @pl.when(jnp.logical_and(cid == SET_CORE[k], c < nch))
            def _():
                pltpu.sync_copy(idxs[k].at[c], ibuf)
                pltpu.sync_copy(msgs[k].at[pl.ds(c * CH, CH)], vbuf)
                pltpu.sync_copy(vbuf, sh.at[ibuf], add=True)
    plsc.subcore_barrier()
    pltpu.sync_copy(sh.at[pl.ds(sid * STRIPE, STRIPE)],
                    out.at[cid].at[pl.ds(sid * STRIPE, STRIPE)])


def _sc_scatter(msgs, idx2ds, width):
    eps = tuple(m.shape[0] for m in msgs)
    mesh = plsc.VectorSubcoreMesh(core_axis_name="c", subcore_axis_name="s",
                                  num_cores=NC, num_subcores=NS)
    zeros = jnp.zeros((STRIPE, width), jnp.float32)
    out = pl.kernel(
        functools.partial(_scatter_body, eps, width),
        out_type=jax.ShapeDtypeStruct((NC, SH, width), jnp.float32),
        mesh=mesh,
        scratch_types=[
            pltpu.VMEM((CH, width), jnp.float32),
            pltpu.VMEM((CH,), jnp.int32),
            pltpu.VMEM_SHARED((SH, width), jnp.float32),
        ],
    )(*msgs, *idx2ds, zeros)
    return out


def _prep_idx(idx, e_pad, off):
    idx = jnp.pad(idx, (0, e_pad - idx.shape[0])) + off
    return idx.reshape(e_pad // CH, CH).astype(jnp.int32)


def _pad_rows(x, e_pad):
    return jnp.pad(x, ((0, e_pad - x.shape[0]), (0, 0)))


def kernel(x_ap, x_user, x_target, ea_s, ea_tx, ea_rx, params, ei_s, ei_tx,
           ei_rx):
    h_ap = _layernorm(_mlp(params["ap_in"], x_ap),
                      params["ln"]["ap"]["g"], params["ln"]["ap"]["b"])
    h_user = _layernorm(_mlp(params["user_in"], x_user),
                        params["ln"]["user"]["g"], params["ln"]["user"]["b"])
    h_tgt = _layernorm(_mlp(params["tgt_in"], x_target),
                       params["ln"]["tgt"]["g"], params["ln"]["tgt"]["b"])

    e_s, e_tx, e_rx = ea_s.shape[0], ea_tx.shape[0], ea_rx.shape[0]
    ep_s = (e_s + BE - 1) // BE * BE
    ep_tx = (e_tx + BE - 1) // BE * BE
    ep_rx = (e_rx + BE - 1) // BE * BE
    ea_s_p = _pad_rows(ea_s, ep_s)
    ea_tx_p = _pad_rows(ea_tx, ep_tx)
    ea_rx_p = _pad_rows(ea_rx, ep_rx)

    n_ap, n_user, n_tgt = x_ap.shape[0], x_user.shape[0], x_target.shape[0]

    idx2ds = (
        _prep_idx(ei_s[1], ep_s, SET_OFF[0]),
        _prep_idx(ei_s[0], ep_s, SET_OFF[1]),
        _prep_idx(ei_tx[0], ep_tx, SET_OFF[2]),
        _prep_idx(ei_rx[0], ep_rx, SET_OFF[3]),
        _prep_idx(ei_tx[1], ep_tx, SET_OFF[4]),
        _prep_idx(ei_rx[1], ep_rx, SET_OFF[5]),
    )

    def ones_masked(e_pad, e_real):
        return jnp.where(jnp.arange(e_pad)[:, None] < e_real,
                         jnp.float32(1), jnp.float32(0)) * jnp.ones((1, HP),
                                                                    jnp.float32)
    cnt = _sc_scatter(
        (ones_masked(ep_s, e_s), ones_masked(ep_s, e_s),
         ones_masked(ep_tx, e_tx), ones_masked(ep_rx, e_rx),
         ones_masked(ep_tx, e_tx), ones_masked(ep_rx, e_rx)),
        idx2ds, HP)
    inv = 1.0 / jnp.maximum(cnt[:, :, :1], 1.0)
    inv_user = inv[0, :n_user]
    inv_ap = (inv[0, NP_USER:NP_USER + n_ap], inv[1, :n_ap],
              inv[1, NP_AP:NP_AP + n_ap])
    inv_tgt = (inv[1, 2 * NP_AP:2 * NP_AP + n_tgt],
               inv[1, 2 * NP_AP + NP_TGT:2 * NP_AP + NP_TGT + n_tgt])

    gidx = (
        _prep_idx(ei_s[0], ep_s, 0), _prep_idx(ei_s[1], ep_s, 0),
        _prep_idx(ei_tx[0], ep_tx, 0), _prep_idx(ei_tx[1], ep_tx, 0),
        _prep_idx(ei_rx[0], ep_rx, 0), _prep_idx(ei_rx[1], ep_rx, 0),
    )

    def pad_tab(h):
        return jnp.pad(h, ((0, 0), (0, HP - H)))

    for lp in params["layers"]:
        c = lp["conv"]
        (xs_ap, xs_user, xtx_ap, xtx_tgt, xrx_ap, xrx_tgt) = _sc_gather(
            (pad_tab(h_ap), pad_tab(h_user), pad_tab(h_tgt)),
            (0, 1, 0, 2, 0, 2), gidx)

        m_s_f, m_s_r = _fused_msgs(ea_s_p, xs_ap, xs_user, lp["net_serv"], e_s)
        m_tx_f, m_tx_r = _fused_msgs(ea_tx_p, xtx_ap, xtx_tgt,
                                     lp["net_sens"], e_tx)
        m_rx_f, m_rx_r = _fused_msgs(ea_rx_p, xrx_ap, xrx_tgt,
                                     lp["net_sens"], e_rx)

        p = _sc_scatter((m_s_f, m_s_r, m_tx_r, m_rx_r, m_tx_f, m_rx_f),
                        idx2ds, HP)[:, :, :H]
        agg_user = p[0, :n_user] * inv_user
        agg_ap = (p[0, NP_USER:NP_USER + n_ap] * inv_ap[0]
                  + p[1, :n_ap] * inv_ap[1]
                  + p[1, NP_AP:NP_AP + n_ap] * inv_ap[2])
        agg_tgt = (p[1, 2 * NP_AP:2 * NP_AP + n_tgt] * inv_tgt[0]
                   + p[1, 2 * NP_AP + NP_TGT:2 * NP_AP + NP_TGT + n_tgt]
                   * inv_tgt[1])

        out_user = agg_user + h_user @ c["serves"]["root"] + c["serves"]["bias"]
        out_ap = (agg_ap
                  + h_ap @ (c["rev_served"]["root"] + c["rev_tx"]["root"]
                            + c["rev_rx"]["root"])
                  + c["rev_served"]["bias"] + c["rev_tx"]["bias"]
                  + c["rev_rx"]["bias"])
        out_tgt = (agg_tgt
                   + h_tgt @ (c["tx"]["root"] + c["rx"]["root"])
                   + c["tx"]["bias"] + c["rx"]["bias"])
        h_ap = jax.nn.relu(out_ap)
        h_user = jax.nn.relu(out_user)
        h_tgt = jax.nn.relu(out_tgt)

    ga, gu = _sc_gather((pad_tab(h_ap), pad_tab(h_user)), (0, 1),
                        (gidx[0], gidx[1]))
    x_feats = jnp.concatenate([ga[:e_s, :H], gu[:e_s, :H], ea_s], axis=-1)
    x_logit = _mlp(params["edge_head"], x_feats)
    tau_logit = _mlp(params["ap_head"], h_ap)
    s_logit = _mlp(params["tgt_head"], h_tgt)
    return x_logit, tau_logit, s_logit
